# Initial kernel scaffold; baseline (speedup 1.0000x reference)
import jax
import jax.numpy as jnp
from jax.experimental import pallas as pl


def kernel(x, edge_index, edge_attr, W1, b1, W2, b2, w_ih, w_hh, b_ih, b_hh):
    raise NotImplementedError("write your pallas kernel here")


# speedup vs baseline: 2.3474x; 2.3474x over previous
import functools

import jax
import jax.numpy as jnp
from jax import lax
from jax.experimental import pallas as pl
from jax.experimental.pallas import tpu as pltpu
from jax.experimental.pallas import tpu_sc as plsc

N = 10000
E = 320000
H = 128

NC = 2
NS = 16
NW = NC * NS
CH = 128
CHUNKS = ((E + NW - 1) // NW + CH - 1) // CH
TPT = CH * CHUNKS
EPAD = TPT * NW
NPAD = 10240
RPT = NPAD // NS


def _pre_body(x_ref, wa_ref, wb_ref, b1_ref, xa_ref, xb_ref):
    xv = x_ref[...]
    dn = (((1,), (1,)), ((), ()))
    xa_ref[...] = lax.dot_general(xv, wa_ref[...], dn,
                                  preferred_element_type=jnp.float32) + b1_ref[...]
    xb_ref[...] = lax.dot_general(xv, wb_ref[...], dn,
                                  preferred_element_type=jnp.float32)


def _tc_pre(x_pad, w1a, w1b, b1_2d):
    blk = NPAD // 8
    return pl.pallas_call(
        _pre_body,
        out_shape=(jax.ShapeDtypeStruct((NPAD, H), jnp.float32),
                   jax.ShapeDtypeStruct((NPAD, H), jnp.float32)),
        grid=(8,),
        in_specs=[pl.BlockSpec((blk, H), lambda i: (i, 0)),
                  pl.BlockSpec((H, H), lambda i: (0, 0)),
                  pl.BlockSpec((H, H), lambda i: (0, 0)),
                  pl.BlockSpec((1, H), lambda i: (0, 0))],
        out_specs=(pl.BlockSpec((blk, H), lambda i: (i, 0)),
                   pl.BlockSpec((blk, H), lambda i: (i, 0))),
    )(x_pad, w1a, w1b, b1_2d)


def _sc_body(xa_hbm, xb_hbm, w1e_hbm, row_hbm, col_hbm, ea_hbm,
             outs_hbm, outd_hbm,
             ridx, cidx, eab, ga, gb, ones, w1eb, sacc, dacc, sem):
    c = lax.axis_index("c")
    s = lax.axis_index("s")
    wid = s * NC + c

    zero16 = jnp.zeros((16,), jnp.float32)
    one16 = jnp.ones((16,), jnp.float32)

    def _zrow(r, carry):
        for v in range(H // 16):
            ga[r, pl.ds(v * 16, 16)] = zero16
        return carry
    lax.fori_loop(0, CH, _zrow, 0)
    for v in range(CH // 16):
        ones[pl.ds(v * 16, 16)] = one16
    pltpu.sync_copy(w1e_hbm, w1eb)

    rbase = s * RPT
    for i in range(RPT // CH):
        pltpu.sync_copy(ga, sacc.at[pl.ds(rbase + i * CH, CH)])
        pltpu.sync_copy(ga.at[0], dacc.at[pl.ds(rbase + i * CH, CH)])
    plsc.subcore_barrier()

    w1v = [w1eb[pl.ds(v * 16, 16)] for v in range(H // 16)]

    ebase = wid * TPT

    def _chunk(k, carry):
        off = ebase + k * CH
        pltpu.sync_copy(row_hbm.at[pl.ds(off, CH)], ridx)
        pltpu.sync_copy(col_hbm.at[pl.ds(off, CH)], cidx)
        pltpu.sync_copy(ea_hbm.at[pl.ds(off, CH)], eab)
        cp1 = pltpu.async_copy(xa_hbm.at[ridx], ga, sem)
        cp2 = pltpu.async_copy(xb_hbm.at[cidx], gb, sem)
        cp1.wait()
        cp2.wait()

        def _grp(jv, icarry):
            ev = eab[pl.ds(jv * 16, 16)]
            for l in range(16):
                e = ev[l]
                j = jv * 16 + l
                for v in range(H // 16):
                    sl = pl.ds(v * 16, 16)
                    t = ga[j, sl] + gb[j, sl] + e * w1v[v]
                    ga[j, sl] = t * (1.0 / (1.0 + jnp.exp(-t)))
            return icarry
        lax.fori_loop(0, CH // 16, _grp, 0)

        pltpu.sync_copy(ga, sacc.at[ridx], add=True)
        pltpu.sync_copy(ones, dacc.at[ridx], add=True)
        return carry
    lax.fori_loop(0, CHUNKS, _chunk, 0)

    plsc.subcore_barrier()

    for i in range(RPT // CH):
        r0 = rbase + i * CH
        pltpu.sync_copy(sacc.at[pl.ds(r0, CH)], ga)
        pltpu.sync_copy(ga, outs_hbm.at[c, pl.ds(r0, CH)])
        pltpu.sync_copy(dacc.at[pl.ds(r0, CH)], eab)
        pltpu.sync_copy(eab, outd_hbm.at[c, pl.ds(r0, CH)])


_sc_edge = pl.kernel(
    _sc_body,
    out_type=(jax.ShapeDtypeStruct((NC, NPAD, H), jnp.float32),
              jax.ShapeDtypeStruct((NC, NPAD), jnp.float32)),
    mesh=plsc.VectorSubcoreMesh(core_axis_name="c", subcore_axis_name="s",
                                num_cores=NC, num_subcores=NS),
    scratch_types=[
        pltpu.VMEM((CH,), jnp.int32),
        pltpu.VMEM((CH,), jnp.int32),
        pltpu.VMEM((CH,), jnp.float32),
        pltpu.VMEM((CH, H), jnp.float32),
        pltpu.VMEM((CH, H), jnp.float32),
        pltpu.VMEM((CH,), jnp.float32),
        pltpu.VMEM((H,), jnp.float32),
        pltpu.VMEM_SHARED((NPAD, H), jnp.float32),
        pltpu.VMEM_SHARED((NPAD,), jnp.float32),
        pltpu.SemaphoreType.DMA,
    ],
)


def _post_body(s0_ref, s1_ref, d0_ref, d1_ref, x_ref, w2_ref, b2_ref,
               wih_ref, whh_ref, bih_ref, bhh_ref, out_ref):
    dn = (((1,), (1,)), ((), ()))
    S = s0_ref[0] + s1_ref[0]
    deg = d0_ref[0] + d1_ref[0]
    agg = lax.dot_general(S, w2_ref[...], dn,
                          preferred_element_type=jnp.float32) + deg * b2_ref[...]
    xv = x_ref[...]
    gi = lax.dot_general(agg, wih_ref[...], dn,
                         preferred_element_type=jnp.float32) + bih_ref[...]
    gh = lax.dot_general(xv, whh_ref[...], dn,
                         preferred_element_type=jnp.float32) + bhh_ref[...]
    r = jax.nn.sigmoid(gi[:, :H] + gh[:, :H])
    z = jax.nn.sigmoid(gi[:, H:2 * H] + gh[:, H:2 * H])
    n = jnp.tanh(gi[:, 2 * H:] + r * gh[:, 2 * H:])
    out_ref[...] = (1.0 - z) * n + z * xv


def _tc_post(partS, partD3, x, w2, b2_2d, wih, whh, bih_2d, bhh_2d):
    B = N // 5
    return pl.pallas_call(
        _post_body,
        out_shape=jax.ShapeDtypeStruct((N, H), jnp.float32),
        grid=(5,),
        in_specs=[pl.BlockSpec((1, B, H), lambda i: (0, i, 0)),
                  pl.BlockSpec((1, B, H), lambda i: (1, i, 0)),
                  pl.BlockSpec((1, B, 1), lambda i: (0, i, 0)),
                  pl.BlockSpec((1, B, 1), lambda i: (1, i, 0)),
                  pl.BlockSpec((B, H), lambda i: (i, 0)),
                  pl.BlockSpec((H, H), lambda i: (0, 0)),
                  pl.BlockSpec((1, H), lambda i: (0, 0)),
                  pl.BlockSpec((3 * H, H), lambda i: (0, 0)),
                  pl.BlockSpec((3 * H, H), lambda i: (0, 0)),
                  pl.BlockSpec((1, 3 * H), lambda i: (0, 0)),
                  pl.BlockSpec((1, 3 * H), lambda i: (0, 0))],
        out_specs=pl.BlockSpec((B, H), lambda i: (i, 0)),
    )(partS, partS, partD3, partD3, x, w2, b2_2d, wih, whh, bih_2d, bhh_2d)


def kernel(x, edge_index, edge_attr, W1, b1, W2, b2, w_ih, w_hh, b_ih, b_hh):
    w1a = W1[:, :H]
    w1b = W1[:, H:2 * H]
    w1e = W1[:, 2 * H]

    x_pad = jnp.concatenate(
        [x, jnp.zeros((NPAD - N, H), jnp.float32)], axis=0)
    xa, xb = _tc_pre(x_pad, w1a, w1b, b1[None, :])

    row = edge_index[0].astype(jnp.int32)
    col = edge_index[1].astype(jnp.int32)
    pad_idx = N + (jnp.arange(EPAD - E, dtype=jnp.int32) % (NPAD - N))
    rowp = jnp.concatenate([row, pad_idx])
    colp = jnp.concatenate([col, pad_idx])
    eap = jnp.concatenate([edge_attr[:, 0],
                           jnp.zeros((EPAD - E,), jnp.float32)])

    partS, partD = _sc_edge(xa, xb, w1e, rowp, colp, eap)

    return _tc_post(partS, partD[:, :, None], x, W2, b2[None, :],
                    w_ih, w_hh, b_ih[None, :], b_hh[None, :])
